# baseline (speedup 1.0000x reference)
import functools

import jax
import jax.numpy as jnp
from jax import lax
from jax.experimental import pallas as pl
from jax.experimental.pallas import tpu as pltpu
from jax.experimental.pallas import tpu_sc as plsc

_W = 7
_TOPK = 8
_DH = 32
_EPS = 1e-06


def _feature_map(x):
    return jnp.where(x > 0, x + 1.0, jnp.exp(x))


def _to_windows(x, d, n, w):
    x = jnp.transpose(x.reshape(d, w * n * w), (1, 0))
    x = x.reshape(w, n, w, d)
    x = jnp.transpose(x, (1, 0, 2, 3))
    return x.reshape(n, w * w, d)


def _from_windows(y, d, n, w):
    y = y.reshape(n, w, w, d)
    y = jnp.transpose(y, (1, 0, 2, 3))
    y = y.reshape(w * n * w, d)
    return jnp.transpose(y, (1, 0)).reshape(d, w, n * w)


def _stage1_body(qref, kref, vref, kv_ref, ks_ref, qm_ref, km_ref, vm_ref):
    d = qref.shape[1]
    n = km_ref.shape[1]
    w = _W
    ws = w * w
    heads = d // _DH

    kw = _to_windows(kref[0, :, 0], d, n, w)
    vw = _to_windows(vref[0, :, 0], d, n, w)
    kq = _feature_map(kw)

    g = lax.dot_general(kq, vw, (((1,), (1,)), ((0,), (0,))),
                        preferred_element_type=jnp.float32)
    blocks = [g[:, h * _DH:(h + 1) * _DH, h * _DH:(h + 1) * _DH]
              for h in range(heads)]
    kv_ref[...] = jnp.concatenate(blocks, axis=1).reshape(
        n, d * _DH)
    ks = jnp.sum(kq, axis=1)
    ks_ref[0] = jnp.concatenate(
        [ks, jnp.zeros((n, 128 - d), jnp.float32)], axis=1)
    km_ref[0] = jnp.mean(kw, axis=1)
    vm_ref[0] = jnp.mean(vw, axis=1)
    qw = _to_windows(qref[0, :, 0], d, n, w)
    qm_ref[0] = jnp.mean(qw, axis=1)


def _stage2_body(nw, qm_ref, km_ref, idx_ref):
    bi = pl.program_id(0)
    rows = qm_ref.shape[1]
    sim = lax.dot_general(qm_ref[0], km_ref[0], (((1,), (1,)), ((), ())),
                          preferred_element_type=jnp.float32)
    iota = lax.broadcasted_iota(jnp.int32, (rows, nw), 1)
    cols = []
    for _ in range(_TOPK):
        mx = jnp.max(sim, axis=1, keepdims=True)
        idx = jnp.min(jnp.where(sim >= mx, iota, jnp.int32(2 ** 30)),
                      axis=1)
        cols.append(idx)
        sim = jnp.where(iota == idx[:, None], jnp.float32(-3.0e38), sim)
    idx_ref[0] = jnp.concatenate(
        [c[:, None] for c in cols], axis=1) + bi * nw


def _stage3_body(km_ref, vm_ref, out_ref):
    d = km_ref.shape[2]
    heads = d // _DH
    kq = _feature_map(km_ref[0])
    g = lax.dot_general(kq, vm_ref[0], (((0,), (0,)), ((), ())),
                        preferred_element_type=jnp.float32)
    blocks = [g[h * _DH:(h + 1) * _DH, h * _DH:(h + 1) * _DH]
              for h in range(heads)]
    kv = jnp.concatenate(blocks, axis=0)
    ks = jnp.sum(kq, axis=0)
    out_ref[0] = jnp.concatenate([kv, ks[:, None]], axis=1)


_NC = 2
_NS = 16


def _sc_gather_sum_body(wpt, kv_hbm, ks_hbm, idx_hbm, okv_hbm, oks_hbm,
                        idx_v, rkv_v, rks_v, okv_v, oks_v, sem, osem):
    wid = lax.axis_index("s") * _NC + lax.axis_index("c")
    base = wid * wpt
    pltpu.sync_copy(idx_hbm.at[pl.ds(base * _TOPK, wpt * _TOPK)], idx_v)
    kv_chunks = kv_hbm.shape[1] // 16
    ks_chunks = ks_hbm.shape[1] // 16
    ngroups = wpt // 8

    def group(gg, _):
        for g8 in range(8):
            sl_idx = idx_v.at[pl.ds(gg * 64 + g8 * 8, _TOPK)]
            cp1 = pltpu.async_copy(kv_hbm.at[sl_idx], rkv_v, sem)
            cp2 = pltpu.async_copy(ks_hbm.at[sl_idx], rks_v, sem)
            cp1.wait()
            cp2.wait()

            def chunk(ci, _c):
                sl = pl.ds(ci * 16, 16)
                s0 = rkv_v[0, sl] + rkv_v[1, sl]
                s1 = rkv_v[2, sl] + rkv_v[3, sl]
                s2 = rkv_v[4, sl] + rkv_v[5, sl]
                s3 = rkv_v[6, sl] + rkv_v[7, sl]
                okv_v[g8, sl] = (s0 + s1) + (s2 + s3)
                return _c

            lax.fori_loop(0, kv_chunks, chunk, 0, unroll=4)

            def kchunk(ci, _c):
                sl = pl.ds(ci * 16, 16)
                s0 = rks_v[0, sl] + rks_v[1, sl]
                s1 = rks_v[2, sl] + rks_v[3, sl]
                s2 = rks_v[4, sl] + rks_v[5, sl]
                s3 = rks_v[6, sl] + rks_v[7, sl]
                oks_v[g8, sl] = (s0 + s1) + (s2 + s3)
                return _c

            lax.fori_loop(0, ks_chunks, kchunk, 0)
        o1 = pltpu.async_copy(okv_v, okv_hbm.at[pl.ds(base + gg * 8, 8)], osem)
        o2 = pltpu.async_copy(oks_v, oks_hbm.at[pl.ds(base + gg * 8, 8)], osem)
        o1.wait()
        o2.wait()
        return _

    lax.fori_loop(0, ngroups, group, 0)


def _sc_gather_sum(table_kv, table_ks, idx_flat):
    nrows, kv_words = table_kv.shape
    ks_words = table_ks.shape[1]
    nw_total = idx_flat.shape[0] // _TOPK
    wpt = nw_total // (_NC * _NS)
    mesh = plsc.VectorSubcoreMesh(core_axis_name="c", subcore_axis_name="s")
    return pl.kernel(
        functools.partial(_sc_gather_sum_body, wpt),
        out_type=(
            jax.ShapeDtypeStruct((nw_total, kv_words), jnp.float32),
            jax.ShapeDtypeStruct((nw_total, ks_words), jnp.float32),
        ),
        mesh=mesh,
        scratch_types=[
            pltpu.VMEM((wpt * _TOPK,), jnp.int32),
            pltpu.VMEM((_TOPK, kv_words), jnp.float32),
            pltpu.VMEM((_TOPK, ks_words), jnp.float32),
            pltpu.VMEM((8, kv_words), jnp.float32),
            pltpu.VMEM((8, ks_words), jnp.float32),
            pltpu.SemaphoreType.DMA,
            pltpu.SemaphoreType.DMA,
        ],
    )(table_kv, table_ks, idx_flat)


def _stage5_body(qref, gkv_ref, gks_ref, mean_ref, out_ref):
    d = qref.shape[1]
    n = gkv_ref.shape[1]
    w = _W
    heads = d // _DH

    qw = _feature_map(_to_windows(qref[0, :, 0], d, n, w))
    mean = mean_ref[0]
    kv = gkv_ref[0].reshape(n, d, _DH) + mean[None, :, :_DH]
    ks = gks_ref[0][:, :d] + mean[None, :, _DH]

    hi = lax.broadcasted_iota(jnp.int32, (d, d), 0) // _DH
    hj = lax.broadcasted_iota(jnp.int32, (d, d), 1) // _DH
    mask = (hi == hj).astype(jnp.float32)

    bd = jnp.tile(kv, (1, 1, heads)) * mask[None]
    ksbd = ks[:, :, None] * mask[None]
    both = jnp.concatenate([bd, ksbd], axis=2)
    res = lax.dot_general(qw, both, (((2,), (1,)), ((0,), (0,))),
                          preferred_element_type=jnp.float32)
    msg = res[:, :, :d] / (res[:, :, d:] + _EPS)
    out_ref[0, :, 0] = _from_windows(msg, d, n, w)


def kernel(q, k, v):
    b, d, H, Wd = q.shape
    w = _W
    m, n = H // w, Wd // w
    nw = m * n

    tkv, tks, qmean, kmean, vmean = pl.pallas_call(
        _stage1_body,
        grid=(b, m),
        in_specs=[
            pl.BlockSpec((1, d, 1, w, Wd), lambda bi, mi: (bi, 0, mi, 0, 0)),
            pl.BlockSpec((1, d, 1, w, Wd), lambda bi, mi: (bi, 0, mi, 0, 0)),
            pl.BlockSpec((1, d, 1, w, Wd), lambda bi, mi: (bi, 0, mi, 0, 0)),
        ],
        out_specs=[
            pl.BlockSpec((n, d * _DH), lambda bi, mi: (bi * m + mi, 0)),
            pl.BlockSpec((1, n, 128), lambda bi, mi: (bi, mi, 0)),
            pl.BlockSpec((1, n, d), lambda bi, mi: (bi, mi, 0)),
            pl.BlockSpec((1, n, d), lambda bi, mi: (bi, mi, 0)),
            pl.BlockSpec((1, n, d), lambda bi, mi: (bi, mi, 0)),
        ],
        out_shape=[
            jax.ShapeDtypeStruct((b * nw, d * _DH), jnp.float32),
            jax.ShapeDtypeStruct((b, nw, 128), jnp.float32),
            jax.ShapeDtypeStruct((b, nw, d), jnp.float32),
            jax.ShapeDtypeStruct((b, nw, d), jnp.float32),
            jax.ShapeDtypeStruct((b, nw, d), jnp.float32),
        ],
    )(q.reshape(b, d, m, w, Wd), k.reshape(b, d, m, w, Wd),
      v.reshape(b, d, m, w, Wd))

    rows_blk = 128
    idx = pl.pallas_call(
        functools.partial(_stage2_body, nw),
        grid=(b, nw // rows_blk),
        in_specs=[
            pl.BlockSpec((1, rows_blk, d), lambda bi, ri: (bi, ri, 0)),
            pl.BlockSpec((1, nw, d), lambda bi, ri: (bi, 0, 0)),
        ],
        out_specs=pl.BlockSpec((1, rows_blk, _TOPK),
                               lambda bi, ri: (bi, ri, 0)),
        out_shape=jax.ShapeDtypeStruct((b, nw, _TOPK), jnp.int32),
    )(qmean, kmean)

    mean_ext = pl.pallas_call(
        _stage3_body,
        grid=(b,),
        in_specs=[
            pl.BlockSpec((1, nw, d), lambda bi: (bi, 0, 0)),
            pl.BlockSpec((1, nw, d), lambda bi: (bi, 0, 0)),
        ],
        out_specs=pl.BlockSpec((1, d, _DH + 1), lambda bi: (bi, 0, 0)),
        out_shape=jax.ShapeDtypeStruct((b, d, _DH + 1), jnp.float32),
    )(kmean, vmean)

    idx_flat = idx.reshape(b * nw * _TOPK)
    gkv, gks = _sc_gather_sum(tkv, tks.reshape(b * nw, 128), idx_flat)

    out = pl.pallas_call(
        _stage5_body,
        grid=(b, m),
        in_specs=[
            pl.BlockSpec((1, d, 1, w, Wd), lambda bi, mi: (bi, 0, mi, 0, 0)),
            pl.BlockSpec((1, n, d * _DH), lambda bi, mi: (bi, mi, 0)),
            pl.BlockSpec((1, n, 128), lambda bi, mi: (bi, mi, 0)),
            pl.BlockSpec((1, d, _DH + 1), lambda bi, mi: (bi, 0, 0)),
        ],
        out_specs=pl.BlockSpec((1, d, 1, w, Wd),
                               lambda bi, mi: (bi, 0, mi, 0, 0)),
        out_shape=jax.ShapeDtypeStruct((b, d, m, w, Wd), jnp.float32),
    )(q.reshape(b, d, m, w, Wd), gkv.reshape(b, nw, d * _DH),
      gks.reshape(b, nw, 128), mean_ext)

    return out.reshape(b, d, H, Wd)


# speedup vs baseline: 1.0928x; 1.0928x over previous
import functools

import jax
import jax.numpy as jnp
from jax import lax
from jax.experimental import pallas as pl
from jax.experimental.pallas import tpu as pltpu
from jax.experimental.pallas import tpu_sc as plsc

_W = 7
_TOPK = 8
_DH = 32
_EPS = 1e-06


def _feature_map(x):
    return jnp.where(x > 0, x + 1.0, jnp.exp(x))


def _to_windows(x, d, n, w):
    x = jnp.transpose(x.reshape(d, w * n * w), (1, 0))
    x = x.reshape(w, n, w, d)
    x = jnp.transpose(x, (1, 0, 2, 3))
    return x.reshape(n, w * w, d)


def _from_windows(y, d, n, w):
    y = y.reshape(n, w, w, d)
    y = jnp.transpose(y, (1, 0, 2, 3))
    y = y.reshape(w * n * w, d)
    return jnp.transpose(y, (1, 0)).reshape(d, w, n * w)


def _stage1_body(qref, kref, vref, tab_ref, qm_ref, km_ref, vm_ref):
    d = qref.shape[1]
    n = km_ref.shape[1]
    w = _W
    ws = w * w
    heads = d // _DH

    kw = _to_windows(kref[0, :, 0], d, n, w)
    vw = _to_windows(vref[0, :, 0], d, n, w)
    kq = _feature_map(kw)

    g = lax.dot_general(kq, vw, (((1,), (1,)), ((0,), (0,))),
                        preferred_element_type=jnp.float32)
    blocks = [g[:, h * _DH:(h + 1) * _DH, h * _DH:(h + 1) * _DH]
              for h in range(heads)]
    ks = jnp.sum(kq, axis=1)
    tab_ref[...] = jnp.concatenate(
        [jnp.concatenate(blocks, axis=1).reshape(n, d * _DH),
         ks, jnp.zeros((n, 128 - d), jnp.float32)], axis=1)
    km_ref[0] = jnp.mean(kw, axis=1)
    vm_ref[0] = jnp.mean(vw, axis=1)
    qw = _to_windows(qref[0, :, 0], d, n, w)
    qm_ref[0] = jnp.mean(qw, axis=1)


def _stage2_body(nw, qm_ref, km_ref, idx_ref):
    bi = pl.program_id(0)
    rows = qm_ref.shape[1]
    sim = lax.dot_general(qm_ref[0], km_ref[0], (((1,), (1,)), ((), ())),
                          preferred_element_type=jnp.float32)
    iota = lax.broadcasted_iota(jnp.int32, (rows, nw), 1)
    cols = []
    for _ in range(_TOPK):
        mx = jnp.max(sim, axis=1, keepdims=True)
        idx = jnp.min(jnp.where(sim >= mx, iota, jnp.int32(2 ** 30)),
                      axis=1)
        cols.append(idx)
        sim = jnp.where(iota == idx[:, None], jnp.float32(-3.0e38), sim)
    idx_ref[0] = jnp.concatenate(
        [c[:, None] for c in cols], axis=1) + bi * nw


def _stage3_body(km_ref, vm_ref, out_ref):
    d = km_ref.shape[2]
    heads = d // _DH
    kq = _feature_map(km_ref[0])
    g = lax.dot_general(kq, vm_ref[0], (((0,), (0,)), ((), ())),
                        preferred_element_type=jnp.float32)
    blocks = [g[h * _DH:(h + 1) * _DH, h * _DH:(h + 1) * _DH]
              for h in range(heads)]
    kv = jnp.concatenate(blocks, axis=0)
    ks = jnp.sum(kq, axis=0)
    out_ref[0] = jnp.concatenate([kv, ks[:, None]], axis=1)


_NC = 2
_NS = 16


def _sc_gather_sum_body(wpt, tab_hbm, idx_hbm, out_hbm,
                        idx_v, r0_v, r1_v, out_v, sem0, sem1, osem):
    wid = lax.axis_index("s") * _NC + lax.axis_index("c")
    base = wid * wpt
    pltpu.sync_copy(idx_hbm.at[pl.ds(base * _TOPK, wpt * _TOPK)], idx_v)
    chunks = tab_hbm.shape[1] // 16
    bufs = (r0_v, r1_v)
    sems = (sem0, sem1)
    ngroups = wpt // 8

    pltpu.async_copy(tab_hbm.at[idx_v.at[pl.ds(0, _TOPK)]], r0_v, sem0)

    def group(gg, _):
        for j in range(8):
            noff = lax.rem(gg * 8 + j + 1, wpt) * _TOPK
            pltpu.async_copy(tab_hbm.at[idx_v.at[pl.ds(noff, _TOPK)]],
                             bufs[(j + 1) % 2], sems[(j + 1) % 2])
            cur = bufs[j % 2]
            pltpu.make_async_copy(tab_hbm.at[idx_v.at[pl.ds(0, _TOPK)]],
                                  cur, sems[j % 2]).wait()

            def chunk(ci, _c):
                sl = pl.ds(ci * 16, 16)
                s0 = cur[0, sl] + cur[1, sl]
                s1 = cur[2, sl] + cur[3, sl]
                s2 = cur[4, sl] + cur[5, sl]
                s3 = cur[6, sl] + cur[7, sl]
                out_v[j, sl] = (s0 + s1) + (s2 + s3)
                return _c

            lax.fori_loop(0, chunks, chunk, 0, unroll=4)
        o = pltpu.async_copy(out_v, out_hbm.at[pl.ds(base + gg * 8, 8)], osem)
        o.wait()
        return _

    lax.fori_loop(0, ngroups, group, 0)
    pltpu.make_async_copy(tab_hbm.at[idx_v.at[pl.ds(0, _TOPK)]],
                          r0_v, sem0).wait()


def _sc_gather_sum(table, idx_flat):
    nrows, words = table.shape
    nw_total = idx_flat.shape[0] // _TOPK
    wpt = nw_total // (_NC * _NS)
    mesh = plsc.VectorSubcoreMesh(core_axis_name="c", subcore_axis_name="s")
    return pl.kernel(
        functools.partial(_sc_gather_sum_body, wpt),
        out_type=jax.ShapeDtypeStruct((nw_total, words), jnp.float32),
        mesh=mesh,
        scratch_types=[
            pltpu.VMEM((wpt * _TOPK,), jnp.int32),
            pltpu.VMEM((_TOPK, words), jnp.float32),
            pltpu.VMEM((_TOPK, words), jnp.float32),
            pltpu.VMEM((8, words), jnp.float32),
            pltpu.SemaphoreType.DMA,
            pltpu.SemaphoreType.DMA,
            pltpu.SemaphoreType.DMA,
        ],
    )(table, idx_flat)


def _stage5_body(qref, gt_ref, mean_ref, out_ref):
    d = qref.shape[1]
    n = gt_ref.shape[1]
    w = _W
    heads = d // _DH

    qw = _feature_map(_to_windows(qref[0, :, 0], d, n, w))
    mean = mean_ref[0]
    kv = (gt_ref[0][:, :d * _DH].reshape(n, d, _DH)
          + mean[None, :, :_DH])
    ks = gt_ref[0][:, d * _DH:d * _DH + d] + mean[None, :, _DH]

    hi = lax.broadcasted_iota(jnp.int32, (d, d), 0) // _DH
    hj = lax.broadcasted_iota(jnp.int32, (d, d), 1) // _DH
    mask = (hi == hj).astype(jnp.float32)

    bd = jnp.tile(kv, (1, 1, heads)) * mask[None]
    ksbd = ks[:, :, None] * mask[None]
    both = jnp.concatenate([bd, ksbd], axis=2)
    res = lax.dot_general(qw, both, (((2,), (1,)), ((0,), (0,))),
                          preferred_element_type=jnp.float32)
    msg = res[:, :, :d] / (res[:, :, d:] + _EPS)
    out_ref[0, :, 0] = _from_windows(msg, d, n, w)


def kernel(q, k, v):
    b, d, H, Wd = q.shape
    w = _W
    m, n = H // w, Wd // w
    nw = m * n

    tw = d * _DH + 128
    tab, qmean, kmean, vmean = pl.pallas_call(
        _stage1_body,
        grid=(b, m),
        in_specs=[
            pl.BlockSpec((1, d, 1, w, Wd), lambda bi, mi: (bi, 0, mi, 0, 0)),
            pl.BlockSpec((1, d, 1, w, Wd), lambda bi, mi: (bi, 0, mi, 0, 0)),
            pl.BlockSpec((1, d, 1, w, Wd), lambda bi, mi: (bi, 0, mi, 0, 0)),
        ],
        out_specs=[
            pl.BlockSpec((n, tw), lambda bi, mi: (bi * m + mi, 0)),
            pl.BlockSpec((1, n, d), lambda bi, mi: (bi, mi, 0)),
            pl.BlockSpec((1, n, d), lambda bi, mi: (bi, mi, 0)),
            pl.BlockSpec((1, n, d), lambda bi, mi: (bi, mi, 0)),
        ],
        out_shape=[
            jax.ShapeDtypeStruct((b * nw, tw), jnp.float32),
            jax.ShapeDtypeStruct((b, nw, d), jnp.float32),
            jax.ShapeDtypeStruct((b, nw, d), jnp.float32),
            jax.ShapeDtypeStruct((b, nw, d), jnp.float32),
        ],
    )(q.reshape(b, d, m, w, Wd), k.reshape(b, d, m, w, Wd),
      v.reshape(b, d, m, w, Wd))

    rows_blk = 128
    idx = pl.pallas_call(
        functools.partial(_stage2_body, nw),
        grid=(b, nw // rows_blk),
        in_specs=[
            pl.BlockSpec((1, rows_blk, d), lambda bi, ri: (bi, ri, 0)),
            pl.BlockSpec((1, nw, d), lambda bi, ri: (bi, 0, 0)),
        ],
        out_specs=pl.BlockSpec((1, rows_blk, _TOPK),
                               lambda bi, ri: (bi, ri, 0)),
        out_shape=jax.ShapeDtypeStruct((b, nw, _TOPK), jnp.int32),
    )(qmean, kmean)

    mean_ext = pl.pallas_call(
        _stage3_body,
        grid=(b,),
        in_specs=[
            pl.BlockSpec((1, nw, d), lambda bi: (bi, 0, 0)),
            pl.BlockSpec((1, nw, d), lambda bi: (bi, 0, 0)),
        ],
        out_specs=pl.BlockSpec((1, d, _DH + 1), lambda bi: (bi, 0, 0)),
        out_shape=jax.ShapeDtypeStruct((b, d, _DH + 1), jnp.float32),
    )(kmean, vmean)

    idx_flat = idx.reshape(b * nw * _TOPK)
    gt = _sc_gather_sum(tab, idx_flat)

    out = pl.pallas_call(
        _stage5_body,
        grid=(b, m),
        in_specs=[
            pl.BlockSpec((1, d, 1, w, Wd), lambda bi, mi: (bi, 0, mi, 0, 0)),
            pl.BlockSpec((1, n, tw), lambda bi, mi: (bi, mi, 0)),
            pl.BlockSpec((1, d, _DH + 1), lambda bi, mi: (bi, 0, 0)),
        ],
        out_specs=pl.BlockSpec((1, d, 1, w, Wd),
                               lambda bi, mi: (bi, 0, mi, 0, 0)),
        out_shape=jax.ShapeDtypeStruct((b, d, m, w, Wd), jnp.float32),
    )(q.reshape(b, d, m, w, Wd), gt.reshape(b, nw, tw), mean_ext)

    return out.reshape(b, d, H, Wd)
